# baseline (speedup 1.0000x reference)
import functools

import jax
import jax.numpy as jnp
from jax import lax
from jax.experimental import pallas as pl
from jax.experimental.pallas import tpu as pltpu
from jax.experimental.pallas import tpu_sc as plsc

NC = 2
NS = 16
NW = NC * NS
L = 16


def _mesh():
    return plsc.VectorSubcoreMesh(
        core_axis_name="c", subcore_axis_name="s", num_cores=NC, num_subcores=NS
    )


@functools.lru_cache(maxsize=None)
def _max_kernel(n):
    chunk = n // NW

    @functools.partial(
        pl.kernel,
        out_type=jax.ShapeDtypeStruct((NW * L,), jnp.float32),
        mesh=_mesh(),
        compiler_params=pltpu.CompilerParams(needs_layout_passes=False),
        scratch_types=[
            pltpu.VMEM((chunk,), jnp.float32),
            pltpu.VMEM((chunk,), jnp.float32),
            pltpu.VMEM((L,), jnp.float32),
        ],
    )
    def kmax(pred_hbm, target_hbm, out_hbm, pred_v, target_v, mx_v):
        wid = lax.axis_index("s") * NC + lax.axis_index("c")
        base = wid * chunk
        pltpu.sync_copy(pred_hbm.at[pl.ds(base, chunk)], pred_v)
        pltpu.sync_copy(target_hbm.at[pl.ds(base, chunk)], target_v)

        def body(i, m):
            p = pred_v[pl.ds(i * L, L)]
            t = target_v[pl.ds(i * L, L)]
            return jnp.maximum(m, jnp.abs(p - t))

        m = lax.fori_loop(0, chunk // L, body, jnp.zeros((L,), jnp.float32))
        mx_v[...] = m
        pltpu.sync_copy(mx_v, out_hbm.at[pl.ds(wid * L, L)])

    return kmax


@functools.lru_cache(maxsize=None)
def _loss_kernel(n, bins):
    chunk = n // NW

    @functools.partial(
        pl.kernel,
        out_type=jax.ShapeDtypeStruct((NW * L,), jnp.float32),
        mesh=_mesh(),
        compiler_params=pltpu.CompilerParams(needs_layout_passes=False),
        scratch_types=[
            pltpu.VMEM((chunk,), jnp.float32),
            pltpu.VMEM((chunk,), jnp.float32),
            pltpu.VMEM((NW * L,), jnp.float32),
            pltpu.VMEM((bins,), jnp.float32),
            pltpu.VMEM((L,), jnp.float32),
            pltpu.VMEM((L,), jnp.float32),
        ],
    )
    def kloss(pred_hbm, target_hbm, pmax_hbm, dens_hbm, out_hbm,
              pred_v, target_v, pmax_v, dens_v, wtab_v, acc_v):
        wid = lax.axis_index("s") * NC + lax.axis_index("c")
        base = wid * chunk
        pltpu.sync_copy(pmax_hbm, pmax_v)
        pltpu.sync_copy(dens_hbm, dens_v)
        pltpu.sync_copy(pred_hbm.at[pl.ds(base, chunk)], pred_v)
        pltpu.sync_copy(target_hbm.at[pl.ds(base, chunk)], target_v)

        def mbody(i, m):
            return jnp.maximum(m, pmax_v[pl.ds(i * L, L)])

        m = lax.fori_loop(0, NW, mbody, jnp.zeros((L,), jnp.float32))
        lane = lax.iota(jnp.int32, L)
        for stride in (8, 4, 2, 1):
            acc_v[...] = m
            m = jnp.maximum(m, plsc.load_gather(acc_v, [lane ^ stride]))
        scale_v = (bins - 1.0) / m

        lane = lax.iota(jnp.int32, L)
        g = plsc.load_gather(dens_v, [jnp.minimum(lane, bins - 1)])
        wtab_v[...] = 1.0 / (g + 1e-6)

        def body(i, acc):
            p = pred_v[pl.ds(i * L, L)]
            t = target_v[pl.ds(i * L, L)]
            d = p - t
            idx = (jnp.abs(d) * scale_v).astype(jnp.int32)
            idx = jnp.clip(idx, 0, bins - 1)
            w = plsc.load_gather(wtab_v, [idx])
            return acc + w * d * d

        acc = lax.fori_loop(0, chunk // L, body, jnp.zeros((L,), jnp.float32))
        acc_v[...] = acc
        pltpu.sync_copy(acc_v, out_hbm.at[pl.ds(wid * L, L)])

    return kloss


def kernel(pred, target, gradient_hist, grad_density):
    del gradient_hist
    n = pred.shape[0]
    bins = grad_density.shape[0]
    pmax = _max_kernel(n)(pred, target)
    parts = _loss_kernel(n, bins)(pred, target, pmax, grad_density)
    return jnp.sum(parts) / n


# speedup vs baseline: 1.2050x; 1.2050x over previous
import functools

import jax
import jax.numpy as jnp
from jax import lax
from jax.experimental import pallas as pl
from jax.experimental.pallas import tpu as pltpu
from jax.experimental.pallas import tpu_sc as plsc

NC = 2
NS = 16
NW = NC * NS
L = 16
U = 4


def _mesh():
    return plsc.VectorSubcoreMesh(
        core_axis_name="c", subcore_axis_name="s", num_cores=NC, num_subcores=NS
    )


@functools.lru_cache(maxsize=None)
def _ghm_kernel(n, bins):
    slice_n = n // NS
    half = slice_n // NC
    groups1 = slice_n // L
    groups2 = half // L

    @functools.partial(
        pl.kernel,
        out_type=jax.ShapeDtypeStruct((NW * L,), jnp.float32),
        mesh=_mesh(),
        compiler_params=pltpu.CompilerParams(needs_layout_passes=False),
        scratch_types=[
            pltpu.VMEM((slice_n,), jnp.float32),
            pltpu.VMEM((slice_n,), jnp.float32),
            pltpu.VMEM_SHARED((NS * L,), jnp.float32),
            pltpu.VMEM((NS * L,), jnp.float32),
            pltpu.VMEM((bins,), jnp.float32),
            pltpu.VMEM((L,), jnp.float32),
            pltpu.VMEM((L,), jnp.float32),
        ],
    )
    def kghm(pred_hbm, target_hbm, dens_hbm, out_hbm,
             pred_v, target_v, pmax_sh, pmax_l, dens_v, wtab_v, mx_v):
        c = lax.axis_index("c")
        s = lax.axis_index("s")
        base = s * slice_n
        pltpu.sync_copy(dens_hbm, dens_v)
        pltpu.sync_copy(pred_hbm.at[pl.ds(base, slice_n)], pred_v)
        pltpu.sync_copy(target_hbm.at[pl.ds(base, slice_n)], target_v)

        z = jnp.zeros((L,), jnp.float32)

        def body1(i, ms):
            o = i * (U * L)
            out = []
            for u in range(U):
                p = pred_v[pl.ds(o + u * L, L)]
                t = target_v[pl.ds(o + u * L, L)]
                out.append(jnp.maximum(ms[u], jnp.abs(p - t)))
            return tuple(out)

        ms = lax.fori_loop(0, groups1 // U, body1, (z,) * U)
        m = jnp.maximum(jnp.maximum(ms[0], ms[1]), jnp.maximum(ms[2], ms[3]))

        mx_v[...] = m
        pltpu.sync_copy(mx_v, pmax_sh.at[pl.ds(s * L, L)])
        plsc.subcore_barrier()
        pltpu.sync_copy(pmax_sh, pmax_l)

        def bodym(i, mm):
            return jnp.maximum(mm, pmax_l[pl.ds(i * L, L)])

        m = lax.fori_loop(0, NS, bodym, z)

        lane = lax.iota(jnp.int32, L)
        for stride in (8, 4, 2, 1):
            mx_v[...] = m
            m = jnp.maximum(m, plsc.load_gather(mx_v, [lane ^ stride]))
        scale_v = (bins - 1.0) / m

        g = plsc.load_gather(dens_v, [jnp.minimum(lane, bins - 1)])
        wtab_v[...] = 1.0 / (g + 1e-6)

        hbase = c * half

        def body2(i, accs):
            o = hbase + i * (U * L)
            out = []
            for u in range(U):
                p = pred_v[pl.ds(o + u * L, L)]
                t = target_v[pl.ds(o + u * L, L)]
                d = p - t
                idx = (jnp.abs(d) * scale_v).astype(jnp.int32)
                idx = jnp.clip(idx, 0, bins - 1)
                w = plsc.load_gather(wtab_v, [idx])
                out.append(accs[u] + w * d * d)
            return tuple(out)

        accs = lax.fori_loop(0, groups2 // U, body2, (z,) * U)
        acc = (accs[0] + accs[1]) + (accs[2] + accs[3])
        mx_v[...] = acc
        wid = s * NC + c
        pltpu.sync_copy(mx_v, out_hbm.at[pl.ds(wid * L, L)])

    return kghm


def kernel(pred, target, gradient_hist, grad_density):
    del gradient_hist
    n = pred.shape[0]
    bins = grad_density.shape[0]
    parts = _ghm_kernel(n, bins)(pred, target, grad_density)
    return jnp.sum(parts) / n


# speedup vs baseline: 1.2627x; 1.0479x over previous
import functools

import jax
import jax.numpy as jnp
from jax import lax
from jax.experimental import pallas as pl
from jax.experimental.pallas import tpu as pltpu
from jax.experimental.pallas import tpu_sc as plsc

NC = 2
NS = 16
NW = NC * NS
L = 16
U = 4


def _mesh():
    return plsc.VectorSubcoreMesh(
        core_axis_name="c", subcore_axis_name="s", num_cores=NC, num_subcores=NS
    )


NB = 4


@functools.lru_cache(maxsize=None)
def _ghm_kernel(n, bins):
    slice_n = n // NS
    half = slice_n // NC
    chunk = slice_n // NB
    groups2 = half // L

    @functools.partial(
        pl.kernel,
        out_type=jax.ShapeDtypeStruct((NW * L,), jnp.float32),
        mesh=_mesh(),
        compiler_params=pltpu.CompilerParams(needs_layout_passes=False),
        scratch_types=[
            pltpu.VMEM((slice_n,), jnp.float32),
            pltpu.VMEM((slice_n,), jnp.float32),
            pltpu.VMEM_SHARED((NS * L,), jnp.float32),
            pltpu.VMEM((NS * L,), jnp.float32),
            pltpu.VMEM((bins,), jnp.float32),
            pltpu.VMEM((L,), jnp.float32),
            pltpu.VMEM((L,), jnp.float32),
        ] + [pltpu.SemaphoreType.DMA] * (2 * NB),
    )
    def kghm(pred_hbm, target_hbm, dens_hbm, out_hbm,
             pred_v, target_v, pmax_sh, pmax_l, dens_v, wtab_v, mx_v, *sems):
        c = lax.axis_index("c")
        s = lax.axis_index("s")
        base = s * slice_n
        copies = []
        for k in range(NB):
            o = k * chunk
            copies.append((
                pltpu.async_copy(pred_hbm.at[pl.ds(base + o, chunk)],
                                 pred_v.at[pl.ds(o, chunk)], sems[2 * k]),
                pltpu.async_copy(target_hbm.at[pl.ds(base + o, chunk)],
                                 target_v.at[pl.ds(o, chunk)], sems[2 * k + 1]),
            ))
        pltpu.sync_copy(dens_hbm, dens_v)

        z = jnp.zeros((L,), jnp.float32)

        def body1(i, ms):
            o = i * (U * L)
            out = []
            for u in range(U):
                p = pred_v[pl.ds(o + u * L, L)]
                t = target_v[pl.ds(o + u * L, L)]
                out.append(jnp.maximum(ms[u], jnp.abs(p - t)))
            return tuple(out)

        ms = (z,) * U
        for k in range(NB):
            copies[k][0].wait()
            copies[k][1].wait()
            lo = k * chunk // (U * L)
            ms = lax.fori_loop(lo, lo + chunk // (U * L), body1, ms)
        m = jnp.maximum(jnp.maximum(ms[0], ms[1]), jnp.maximum(ms[2], ms[3]))

        mx_v[...] = m
        pltpu.sync_copy(mx_v, pmax_sh.at[pl.ds(s * L, L)])
        plsc.subcore_barrier()
        pltpu.sync_copy(pmax_sh, pmax_l)

        def bodym(i, mm):
            return jnp.maximum(mm, pmax_l[pl.ds(i * L, L)])

        m = lax.fori_loop(0, NS, bodym, z)

        lane = lax.iota(jnp.int32, L)
        for stride in (8, 4, 2, 1):
            mx_v[...] = m
            m = jnp.maximum(m, plsc.load_gather(mx_v, [lane ^ stride]))
        scale_v = (bins - 1.0) / m

        g = plsc.load_gather(dens_v, [jnp.minimum(lane, bins - 1)])
        wtab_v[...] = 1.0 / (g + 1e-6)

        hbase = c * half

        def body2(i, accs):
            o = hbase + i * (U * L)
            out = []
            for u in range(U):
                p = pred_v[pl.ds(o + u * L, L)]
                t = target_v[pl.ds(o + u * L, L)]
                d = p - t
                idx = (jnp.abs(d) * scale_v).astype(jnp.int32)
                idx = jnp.clip(idx, 0, bins - 1)
                w = plsc.load_gather(wtab_v, [idx])
                out.append(accs[u] + w * d * d)
            return tuple(out)

        accs = lax.fori_loop(0, groups2 // U, body2, (z,) * U)
        acc = (accs[0] + accs[1]) + (accs[2] + accs[3])
        mx_v[...] = acc
        wid = s * NC + c
        pltpu.sync_copy(mx_v, out_hbm.at[pl.ds(wid * L, L)])

    return kghm


def kernel(pred, target, gradient_hist, grad_density):
    del gradient_hist
    n = pred.shape[0]
    bins = grad_density.shape[0]
    parts = _ghm_kernel(n, bins)(pred, target, grad_density)
    return jnp.sum(parts) / n


# speedup vs baseline: 1.3043x; 1.0329x over previous
import functools

import jax
import jax.numpy as jnp
from jax import lax
from jax.experimental import pallas as pl
from jax.experimental.pallas import tpu as pltpu
from jax.experimental.pallas import tpu_sc as plsc

NC = 2
NS = 16
NW = NC * NS
L = 16
U = 4


def _mesh():
    return plsc.VectorSubcoreMesh(
        core_axis_name="c", subcore_axis_name="s", num_cores=NC, num_subcores=NS
    )


NB = 4


@functools.lru_cache(maxsize=None)
def _ghm_kernel(n, bins):
    slice_n = n // NS
    half = slice_n // NC
    chunk = slice_n // NB
    groups2 = half // L

    @functools.partial(
        pl.kernel,
        out_type=jax.ShapeDtypeStruct((1,), jnp.float32),
        mesh=_mesh(),
        compiler_params=pltpu.CompilerParams(needs_layout_passes=False),
        scratch_types=[
            pltpu.VMEM((slice_n,), jnp.float32),
            pltpu.VMEM((slice_n,), jnp.float32),
            pltpu.VMEM_SHARED((NS * L,), jnp.float32),
            pltpu.VMEM((NS * L,), jnp.float32),
            pltpu.VMEM((bins,), jnp.float32),
            pltpu.VMEM((L,), jnp.float32),
            pltpu.VMEM((L,), jnp.float32),
        ] + [pltpu.SemaphoreType.DMA] * (2 * NB),
    )
    def kghm(pred_hbm, target_hbm, dens_hbm, out_hbm,
             pred_v, target_v, pmax_sh, pmax_l, dens_v, wtab_v, mx_v, *sems):
        c = lax.axis_index("c")
        s = lax.axis_index("s")
        base = s * slice_n
        copies = []
        for k in range(NB):
            o = k * chunk
            copies.append((
                pltpu.async_copy(pred_hbm.at[pl.ds(base + o, chunk)],
                                 pred_v.at[pl.ds(o, chunk)], sems[2 * k]),
                pltpu.async_copy(target_hbm.at[pl.ds(base + o, chunk)],
                                 target_v.at[pl.ds(o, chunk)], sems[2 * k + 1]),
            ))
        pltpu.sync_copy(dens_hbm, dens_v)

        z = jnp.zeros((L,), jnp.float32)

        def body1(i, ms):
            o = i * (U * L)
            out = []
            for u in range(U):
                p = pred_v[pl.ds(o + u * L, L)]
                t = target_v[pl.ds(o + u * L, L)]
                out.append(jnp.maximum(ms[u], jnp.abs(p - t)))
            return tuple(out)

        ms = (z,) * U
        for k in range(NB):
            copies[k][0].wait()
            copies[k][1].wait()
            lo = k * chunk // (U * L)
            ms = lax.fori_loop(lo, lo + chunk // (U * L), body1, ms)
        m = jnp.maximum(jnp.maximum(ms[0], ms[1]), jnp.maximum(ms[2], ms[3]))

        mx_v[...] = m
        pltpu.sync_copy(mx_v, pmax_sh.at[pl.ds(s * L, L)])
        plsc.subcore_barrier()
        pltpu.sync_copy(pmax_sh, pmax_l)

        def bodym(i, mm):
            return jnp.maximum(mm, pmax_l[pl.ds(i * L, L)])

        m = lax.fori_loop(0, NS, bodym, z)

        lane = lax.iota(jnp.int32, L)
        for stride in (8, 4, 2, 1):
            mx_v[...] = m
            m = jnp.maximum(m, plsc.load_gather(mx_v, [lane ^ stride]))
        scale_v = (bins - 1.0) / m

        g = plsc.load_gather(dens_v, [jnp.minimum(lane, bins - 1)])
        wtab_v[...] = 1.0 / (g + 1e-6)

        def body2(i, accs):
            o = i * (U * L)
            out = []
            for u in range(U):
                p = pred_v[pl.ds(o + u * L, L)]
                t = target_v[pl.ds(o + u * L, L)]
                d = p - t
                idx = (jnp.abs(d) * scale_v).astype(jnp.int32)
                idx = jnp.clip(idx, 0, bins - 1)
                w = plsc.load_gather(wtab_v, [idx])
                out.append(accs[u] + w * d * d)
            return tuple(out)

        accs = lax.fori_loop(0, NC * groups2 // U, body2, (z,) * U)
        acc = (accs[0] + accs[1]) + (accs[2] + accs[3])

        mx_v[...] = acc
        pltpu.sync_copy(mx_v, pmax_sh.at[pl.ds(s * L, L)])
        plsc.subcore_barrier()

        @pl.when(jnp.logical_and(c == 0, s == 0))
        def _():
            pltpu.sync_copy(pmax_sh, pmax_l)

            def bodys(i, a):
                return a + pmax_l[pl.ds(i * L, L)]

            a = lax.fori_loop(0, NS, bodys, z)
            for stride in (8, 4, 2, 1):
                wtab_v[...] = a
                a = a + plsc.load_gather(wtab_v, [lane ^ stride])
            mx_v[...] = a * (1.0 / n)
            pltpu.sync_copy(mx_v.at[pl.ds(0, 1)], out_hbm)

    return kghm


def kernel(pred, target, gradient_hist, grad_density):
    del gradient_hist
    n = pred.shape[0]
    bins = grad_density.shape[0]
    parts = _ghm_kernel(n, bins)(pred, target, grad_density)
    return parts.reshape(())


# speedup vs baseline: 1.3074x; 1.0024x over previous
import functools

import jax
import jax.numpy as jnp
from jax import lax
from jax.experimental import pallas as pl
from jax.experimental.pallas import tpu as pltpu
from jax.experimental.pallas import tpu_sc as plsc

NC = 2
NS = 16
NW = NC * NS
L = 16
U = 4
U2 = 8


def _mesh():
    return plsc.VectorSubcoreMesh(
        core_axis_name="c", subcore_axis_name="s", num_cores=NC, num_subcores=NS
    )


NB = 4


@functools.lru_cache(maxsize=None)
def _ghm_kernel(n, bins):
    slice_n = n // NS
    half = slice_n // NC
    chunk = slice_n // NB
    groups2 = half // L

    @functools.partial(
        pl.kernel,
        out_type=jax.ShapeDtypeStruct((1,), jnp.float32),
        mesh=_mesh(),
        compiler_params=pltpu.CompilerParams(needs_layout_passes=False),
        scratch_types=[
            pltpu.VMEM((slice_n,), jnp.float32),
            pltpu.VMEM((slice_n,), jnp.float32),
            pltpu.VMEM_SHARED((NS * L,), jnp.float32),
            pltpu.VMEM((NS * L,), jnp.float32),
            pltpu.VMEM((bins,), jnp.float32),
            pltpu.VMEM((L,), jnp.float32),
            pltpu.VMEM((L,), jnp.float32),
        ] + [pltpu.SemaphoreType.DMA] * (2 * NB),
    )
    def kghm(pred_hbm, target_hbm, dens_hbm, out_hbm,
             pred_v, target_v, pmax_sh, pmax_l, dens_v, wtab_v, mx_v, *sems):
        c = lax.axis_index("c")
        s = lax.axis_index("s")
        base = s * slice_n
        copies = []
        for k in range(NB):
            o = k * chunk
            copies.append((
                pltpu.async_copy(pred_hbm.at[pl.ds(base + o, chunk)],
                                 pred_v.at[pl.ds(o, chunk)], sems[2 * k]),
                pltpu.async_copy(target_hbm.at[pl.ds(base + o, chunk)],
                                 target_v.at[pl.ds(o, chunk)], sems[2 * k + 1]),
            ))
        pltpu.sync_copy(dens_hbm, dens_v)

        z = jnp.zeros((L,), jnp.float32)

        def body1(i, ms):
            o = i * (U * L)
            out = []
            for u in range(U):
                p = pred_v[pl.ds(o + u * L, L)]
                t = target_v[pl.ds(o + u * L, L)]
                out.append(jnp.maximum(ms[u], jnp.abs(p - t)))
            return tuple(out)

        ms = (z,) * U
        for k in range(NB):
            copies[k][0].wait()
            copies[k][1].wait()
            lo = k * chunk // (U * L)
            ms = lax.fori_loop(lo, lo + chunk // (U * L), body1, ms)
        m = jnp.maximum(jnp.maximum(ms[0], ms[1]), jnp.maximum(ms[2], ms[3]))

        mx_v[...] = m
        pltpu.sync_copy(mx_v, pmax_sh.at[pl.ds(s * L, L)])
        plsc.subcore_barrier()
        pltpu.sync_copy(pmax_sh, pmax_l)

        def bodym(i, mm):
            return jnp.maximum(mm, pmax_l[pl.ds(i * L, L)])

        m = lax.fori_loop(0, NS, bodym, z)

        lane = lax.iota(jnp.int32, L)
        for stride in (8, 4, 2, 1):
            mx_v[...] = m
            m = jnp.maximum(m, plsc.load_gather(mx_v, [lane ^ stride]))
        scale_v = (bins - 1.0) / m

        g = plsc.load_gather(dens_v, [jnp.minimum(lane, bins - 1)])
        wtab_v[...] = 1.0 / (g + 1e-6)

        def body2(i, accs):
            o = i * (U2 * L)
            out = []
            for u in range(U2):
                p = pred_v[pl.ds(o + u * L, L)]
                t = target_v[pl.ds(o + u * L, L)]
                d = p - t
                idx = (jnp.abs(d) * scale_v).astype(jnp.int32)
                idx = jnp.clip(idx, 0, bins - 1)
                w = plsc.load_gather(wtab_v, [idx])
                out.append(accs[u] + w * d * d)
            return tuple(out)

        accs = lax.fori_loop(0, NC * groups2 // U2, body2, (z,) * U2)
        acc = z
        for u in range(U2):
            acc = acc + accs[u]

        mx_v[...] = acc
        pltpu.sync_copy(mx_v, pmax_sh.at[pl.ds(s * L, L)])
        plsc.subcore_barrier()

        @pl.when(jnp.logical_and(c == 0, s == 0))
        def _():
            pltpu.sync_copy(pmax_sh, pmax_l)

            def bodys(i, a):
                return a + pmax_l[pl.ds(i * L, L)]

            a = lax.fori_loop(0, NS, bodys, z)
            for stride in (8, 4, 2, 1):
                wtab_v[...] = a
                a = a + plsc.load_gather(wtab_v, [lane ^ stride])
            mx_v[...] = a * (1.0 / n)
            pltpu.sync_copy(mx_v.at[pl.ds(0, 1)], out_hbm)

    return kghm


def kernel(pred, target, gradient_hist, grad_density):
    del gradient_hist
    n = pred.shape[0]
    bins = grad_density.shape[0]
    parts = _ghm_kernel(n, bins)(pred, target, grad_density)
    return parts.reshape(())
